# Initial kernel scaffold; baseline (speedup 1.0000x reference)
import jax
import jax.numpy as jnp
from jax.experimental import pallas as pl


def kernel(triplets, unseen_entity, entity_embedding, basis, att):
    raise NotImplementedError("write your pallas kernel here")


# speedup vs baseline: 9.1805x; 9.1805x over previous
import functools

import jax
import jax.numpy as jnp
from jax import lax
from jax.experimental import pallas as pl
from jax.experimental.pallas import tpu as pltpu
from jax.experimental.pallas import tpu_sc as plsc

_NUM_ENTITIES = 10000
_NUM_RELATIONS = 2000
_DIM = 128
_NUM_BASES = 4
_NUM_TRIPLETS = 16000

_NC = 2
_NS = 16
_NW = _NC * _NS
_LANES = 16
_CHUNK = _NUM_TRIPLETS // _NW
_GROUPS = -(-_CHUNK // _LANES)
_ATT_FLAT = 2 * _NUM_RELATIONS * _NUM_BASES


def _sc_body(trip_hbm, u_hbm, att_hbm, ent_hbm, s_out, cnt_out,
             trip_v, u_v, att_v, idx_v, rows_v, s_v, cnt_v, sem):
    wid = lax.axis_index("s") * _NC + lax.axis_index("c")
    lane = lax.iota(jnp.int32, _LANES)

    pltpu.sync_copy(trip_hbm.at[wid], trip_v)
    pltpu.sync_copy(u_hbm, u_v)
    pltpu.sync_copy(att_hbm, att_v)

    zeros16 = jnp.zeros((_LANES,), jnp.float32)
    for b in range(_NUM_BASES):
        for j in range(_DIM // _LANES):
            s_v[b, pl.ds(j * _LANES, _LANES)] = zeros16

    u_vec = u_v[...]

    def accumulate(src_idx, arow, mask):
        idx_v[...] = jnp.where(mask, src_idx, 0)
        pltpu.async_copy(ent_hbm.at[idx_v], rows_v, sem).wait()
        for b in range(_NUM_BASES):
            a_b = plsc.load_gather(att_v, [arow * _NUM_BASES + b])
            a_b = jnp.where(mask, a_b, 0.0)
            ams = [jnp.sum(jnp.where(lane == m, a_b, 0.0))
                   for m in range(_LANES)]
            for j in range(_DIM // _LANES):
                acc = zeros16
                for m in range(_LANES):
                    acc = acc + ams[m] * rows_v[m, pl.ds(j * _LANES, _LANES)]
                plsc.addupdate(s_v.at[b, pl.ds(j * _LANES, _LANES)], acc)

    def group(g, cnt_vec):
        e = g * _LANES + lane
        valid = e < _CHUNK
        e_c = jnp.minimum(e, _CHUNK - 1) * 3
        s = plsc.load_gather(trip_v, [e_c])
        r = plsc.load_gather(trip_v, [e_c + 1])
        d = plsc.load_gather(trip_v, [e_c + 2])
        m1 = (d == u_vec) & valid
        m2 = (s == u_vec) & valid
        cnt_vec = cnt_vec + jnp.where(m1, 1.0, 0.0) + jnp.where(m2, 1.0, 0.0)

        @pl.when(jnp.any(m1))
        def _():
            accumulate(s, r, m1)

        @pl.when(jnp.any(m2))
        def _():
            accumulate(d, r + _NUM_RELATIONS, m2)

        return cnt_vec

    cnt_vec = lax.fori_loop(0, _GROUPS, group, jnp.zeros((_LANES,), jnp.float32))
    cnt_v[...] = cnt_vec

    for b in range(_NUM_BASES):
        pltpu.sync_copy(s_v.at[b], s_out.at[b * _NW + wid])
    pltpu.sync_copy(cnt_v, cnt_out.at[wid])


_sc_kernel = pl.kernel(
    _sc_body,
    out_type=[
        jax.ShapeDtypeStruct((_NUM_BASES * _NW, _DIM), jnp.float32),
        jax.ShapeDtypeStruct((_NW, _LANES), jnp.float32),
    ],
    mesh=plsc.VectorSubcoreMesh(
        core_axis_name="c", subcore_axis_name="s",
        num_cores=_NC, num_subcores=_NS),
    scratch_types=[
        pltpu.VMEM((_CHUNK * 3,), jnp.int32),
        pltpu.VMEM((_LANES,), jnp.int32),
        pltpu.VMEM((_ATT_FLAT,), jnp.float32),
        pltpu.VMEM((_LANES,), jnp.int32),
        pltpu.VMEM((_LANES, _DIM), jnp.float32),
        pltpu.VMEM((_NUM_BASES, _DIM), jnp.float32),
        pltpu.VMEM((_LANES,), jnp.float32),
        pltpu.SemaphoreType.DMA,
    ],
    compiler_params=pltpu.CompilerParams(needs_layout_passes=False),
)


def _tc_body(s_ref, cnt_ref, basis_ref, out_ref):
    cnt = jnp.sum(cnt_ref[...])
    acc = jnp.zeros((1, _DIM), jnp.float32)
    for b in range(_NUM_BASES):
        sb = jnp.sum(s_ref[pl.ds(b * _NW, _NW), :], axis=0, keepdims=True)
        acc = acc + jnp.dot(sb, basis_ref[b],
                            preferred_element_type=jnp.float32)
    out_ref[...] = acc / jnp.maximum(cnt, 1.0)


@jax.jit
def kernel(triplets, unseen_entity, entity_embedding, basis, att):
    trip = triplets.astype(jnp.int32).reshape(_NW, _CHUNK * 3)
    u_splat = jnp.full((_LANES,), unseen_entity, dtype=jnp.int32)
    att_flat = att.reshape(_ATT_FLAT)
    s_all, cnt_all = _sc_kernel(trip, u_splat, att_flat, entity_embedding)
    out = pl.pallas_call(
        _tc_body,
        out_shape=jax.ShapeDtypeStruct((1, _DIM), jnp.float32),
    )(s_all, cnt_all, basis)
    return out.reshape(_DIM)


# speedup vs baseline: 13.4926x; 1.4697x over previous
import jax
import jax.numpy as jnp
from jax import lax
from jax.experimental import pallas as pl
from jax.experimental.pallas import tpu as pltpu
from jax.experimental.pallas import tpu_sc as plsc

_NUM_ENTITIES = 10000
_NUM_RELATIONS = 2000
_DIM = 128
_NUM_BASES = 4
_NUM_TRIPLETS = 16000

_NC = 2
_NS = 16
_NW = _NC * _NS
_LANES = 16
_CHUNK = _NUM_TRIPLETS // _NW
_GROUPS = -(-_CHUNK // _LANES)
_CHUNK_PAD = _GROUPS * _LANES
_SFLAT = _NUM_BASES * _DIM


def _sc_body(trip_hbm, u_hbm, att_hbm, ent_hbm, s_out, cnt_out,
             src_v, rel_v, dst_v, u_v, att_v, idx_v, rows_v, s_v, cnt_v, sem):
    wid = lax.axis_index("s") * _NC + lax.axis_index("c")
    lane = lax.iota(jnp.int32, _LANES)

    copies = [
        pltpu.async_copy(trip_hbm.at[0, wid], src_v, sem),
        pltpu.async_copy(trip_hbm.at[1, wid], rel_v, sem),
        pltpu.async_copy(trip_hbm.at[2, wid], dst_v, sem),
        pltpu.async_copy(u_hbm, u_v, sem),
        pltpu.async_copy(att_hbm, att_v, sem),
    ]

    zeros16 = jnp.zeros((_LANES,), jnp.float32)
    for j in range(_SFLAT // _LANES):
        s_v[pl.ds(j * _LANES, _LANES)] = zeros16
    cnt_v[...] = zeros16

    for c in copies:
        c.wait()
    u_vec = u_v[...]

    def accumulate(gidx, arow, mask):
        plsc.addupdate(cnt_v.at[pl.ds(0, _LANES)], jnp.where(mask, 1.0, 0.0))
        idx_v[...] = jnp.where(mask, gidx, 0)
        pltpu.async_copy(ent_hbm.at[idx_v], rows_v, sem).wait()
        arow_safe = jnp.where(mask, arow, 0)
        for b in range(_NUM_BASES):
            a_b = plsc.load_gather(
                att_v, [jnp.full((_LANES,), b, jnp.int32), arow_safe])
            a_b = jnp.where(mask, a_b, 0.0)

            def mbody(m, accs):
                am = jnp.sum(jnp.where(lane == m, a_b, 0.0))
                return tuple(
                    accs[j] + am * rows_v[m, pl.ds(j * _LANES, _LANES)]
                    for j in range(_DIM // _LANES))

            accs = lax.fori_loop(0, _LANES, mbody,
                                 (zeros16,) * (_DIM // _LANES))
            for j in range(_DIM // _LANES):
                plsc.addupdate(
                    s_v.at[pl.ds((b * 8 + j) * _LANES, _LANES)], accs[j])

    def group(g, carry):
        base = pl.multiple_of(g * _LANES, _LANES)
        s = src_v[pl.ds(base, _LANES)]
        r = rel_v[pl.ds(base, _LANES)]
        d = dst_v[pl.ds(base, _LANES)]
        m1 = d == u_vec
        m2 = s == u_vec

        @pl.when(jnp.any(m1 | m2))
        def _():
            def half(h, c2):
                mask = jnp.where(h == 0, m1, m2)
                gidx = jnp.where(h == 0, s, d)
                arow = jnp.where(h == 0, r, r + _NUM_RELATIONS)

                @pl.when(jnp.any(mask))
                def _():
                    accumulate(gidx, arow, mask)

                return c2

            lax.fori_loop(0, 2, half, 0)

        return carry

    lax.fori_loop(0, _GROUPS, group, 0)

    pltpu.sync_copy(s_v, s_out.at[wid])
    pltpu.sync_copy(cnt_v, cnt_out.at[wid])


_sc_kernel = pl.kernel(
    _sc_body,
    out_type=[
        jax.ShapeDtypeStruct((_NW, _SFLAT), jnp.float32),
        jax.ShapeDtypeStruct((_NW, _LANES), jnp.float32),
    ],
    mesh=plsc.VectorSubcoreMesh(
        core_axis_name="c", subcore_axis_name="s",
        num_cores=_NC, num_subcores=_NS),
    scratch_types=[
        pltpu.VMEM((_CHUNK_PAD,), jnp.int32),
        pltpu.VMEM((_CHUNK_PAD,), jnp.int32),
        pltpu.VMEM((_CHUNK_PAD,), jnp.int32),
        pltpu.VMEM((_LANES,), jnp.int32),
        pltpu.VMEM((_NUM_BASES, 2 * _NUM_RELATIONS), jnp.float32),
        pltpu.VMEM((_LANES,), jnp.int32),
        pltpu.VMEM((_LANES, _DIM), jnp.float32),
        pltpu.VMEM((_SFLAT,), jnp.float32),
        pltpu.VMEM((_LANES,), jnp.float32),
        pltpu.SemaphoreType.DMA,
    ],
    compiler_params=pltpu.CompilerParams(needs_layout_passes=False),
)


def _tc_body(s_ref, cnt_ref, basis_ref, out_ref):
    cnt = jnp.sum(cnt_ref[...])
    s_sum = jnp.sum(s_ref[...], axis=0, keepdims=True)
    acc = jnp.zeros((1, _DIM), jnp.float32)
    for b in range(_NUM_BASES):
        sb = s_sum[:, b * _DIM:(b + 1) * _DIM]
        acc = acc + jnp.dot(sb, basis_ref[b],
                            preferred_element_type=jnp.float32)
    out_ref[...] = acc / jnp.maximum(cnt, 1.0)


@jax.jit
def kernel(triplets, unseen_entity, entity_embedding, basis, att):
    trip_t = triplets.astype(jnp.int32).T.reshape(3, _NW, _CHUNK)
    trip_t = jnp.pad(trip_t, ((0, 0), (0, 0), (0, _CHUNK_PAD - _CHUNK)),
                     constant_values=-1)
    u_splat = jnp.full((_LANES,), unseen_entity, dtype=jnp.int32)
    att_t = att.T
    s_all, cnt_all = _sc_kernel(trip_t, u_splat, att_t, entity_embedding)
    out = pl.pallas_call(
        _tc_body,
        out_shape=jax.ShapeDtypeStruct((1, _DIM), jnp.float32),
    )(s_all, cnt_all, basis)
    return out.reshape(_DIM)


# speedup vs baseline: 14.8284x; 1.0990x over previous
import jax
import jax.numpy as jnp
from jax import lax
from jax.experimental import pallas as pl
from jax.experimental.pallas import tpu as pltpu
from jax.experimental.pallas import tpu_sc as plsc

_NUM_ENTITIES = 10000
_NUM_RELATIONS = 2000
_DIM = 128
_NUM_BASES = 4
_NUM_TRIPLETS = 16000

_NC = 2
_NS = 16
_NW = _NC * _NS
_LANES = 16
_CHUNK = _NUM_TRIPLETS // _NW
_GROUPS = -(-_CHUNK // _LANES)
_CHUNK_PAD = _GROUPS * _LANES
_SFLAT = _NUM_BASES * _DIM
_MATCH_CAP = 2 * _CHUNK_PAD + _LANES


def _sc_body(trip_hbm, u_hbm, att_hbm, ent_hbm, s_out, cnt_out,
             src_v, rel_v, dst_v, u_v, gidx_v, arow_v,
             idx_v, rows_v, att_v, s_v, cnt_v, sem):
    wid = lax.axis_index("s") * _NC + lax.axis_index("c")
    lane = lax.iota(jnp.int32, _LANES)

    copies = [
        pltpu.async_copy(trip_hbm.at[0, wid], src_v, sem),
        pltpu.async_copy(trip_hbm.at[1, wid], rel_v, sem),
        pltpu.async_copy(trip_hbm.at[2, wid], dst_v, sem),
        pltpu.async_copy(u_hbm, u_v, sem),
    ]
    att_cp = pltpu.async_copy(att_hbm, att_v, sem)

    zeros16 = jnp.zeros((_LANES,), jnp.float32)
    for j in range(_SFLAT // _LANES):
        s_v[pl.ds(j * _LANES, _LANES)] = zeros16

    for c in copies:
        c.wait()
    u_vec = u_v[...]

    def group(g, n):
        base = pl.multiple_of(g * _LANES, _LANES)
        s = src_v[pl.ds(base, _LANES)]
        r = rel_v[pl.ds(base, _LANES)]
        d = dst_v[pl.ds(base, _LANES)]
        m1 = d == u_vec
        m2 = s == u_vec
        plsc.store_compressed(gidx_v.at[pl.ds(n, _LANES)], s, mask=m1)
        plsc.store_compressed(arow_v.at[pl.ds(n, _LANES)], r, mask=m1)
        n = n + jnp.sum(m1.astype(jnp.int32))
        plsc.store_compressed(gidx_v.at[pl.ds(n, _LANES)], d, mask=m2)
        plsc.store_compressed(arow_v.at[pl.ds(n, _LANES)],
                              r + _NUM_RELATIONS, mask=m2)
        n = n + jnp.sum(m2.astype(jnp.int32))
        return n

    n = lax.fori_loop(0, _GROUPS, group, jnp.int32(0))
    cnt_v[...] = jnp.full((_LANES,), n).astype(jnp.float32)
    att_cp.wait()

    def batch(i, carry):
        base = pl.multiple_of(i * _LANES, _LANES)
        mask = (base + lane) < n
        gidx = jnp.where(mask, gidx_v[pl.ds(base, _LANES)], 0)
        arow = jnp.where(mask, arow_v[pl.ds(base, _LANES)], 0)
        idx_v[...] = gidx
        pltpu.async_copy(ent_hbm.at[idx_v], rows_v, sem).wait()
        for b in range(_NUM_BASES):
            a_b = plsc.load_gather(
                att_v, [jnp.full((_LANES,), b, jnp.int32), arow])
            a_b = jnp.where(mask, a_b, 0.0)

            def mbody(m, accs):
                am = jnp.sum(jnp.where(lane == m, a_b, 0.0))
                return tuple(
                    accs[j] + am * rows_v[m, pl.ds(j * _LANES, _LANES)]
                    for j in range(_DIM // _LANES))

            accs = lax.fori_loop(0, _LANES, mbody,
                                 (zeros16,) * (_DIM // _LANES))
            for j in range(_DIM // _LANES):
                plsc.addupdate(
                    s_v.at[pl.ds((b * 8 + j) * _LANES, _LANES)], accs[j])
        return carry

    lax.fori_loop(0, (n + _LANES - 1) // _LANES, batch, 0)

    pltpu.sync_copy(s_v, s_out.at[wid])
    pltpu.sync_copy(cnt_v, cnt_out.at[wid])


_sc_kernel = pl.kernel(
    _sc_body,
    out_type=[
        jax.ShapeDtypeStruct((_NW, _SFLAT), jnp.float32),
        jax.ShapeDtypeStruct((_NW, _LANES), jnp.float32),
    ],
    mesh=plsc.VectorSubcoreMesh(
        core_axis_name="c", subcore_axis_name="s",
        num_cores=_NC, num_subcores=_NS),
    scratch_types=[
        pltpu.VMEM((_CHUNK_PAD,), jnp.int32),
        pltpu.VMEM((_CHUNK_PAD,), jnp.int32),
        pltpu.VMEM((_CHUNK_PAD,), jnp.int32),
        pltpu.VMEM((_LANES,), jnp.int32),
        pltpu.VMEM((_MATCH_CAP,), jnp.int32),
        pltpu.VMEM((_MATCH_CAP,), jnp.int32),
        pltpu.VMEM((_LANES,), jnp.int32),
        pltpu.VMEM((_LANES, _DIM), jnp.float32),
        pltpu.VMEM((_NUM_BASES, 2 * _NUM_RELATIONS), jnp.float32),
        pltpu.VMEM((_SFLAT,), jnp.float32),
        pltpu.VMEM((_LANES,), jnp.float32),
        pltpu.SemaphoreType.DMA,
    ],
    compiler_params=pltpu.CompilerParams(needs_layout_passes=False),
)


def _tc_body(s_ref, cnt_ref, basis_ref, out_ref):
    cnt = jnp.sum(cnt_ref[:, :1])
    s_sum = jnp.sum(s_ref[...], axis=0, keepdims=True)
    acc = jnp.zeros((1, _DIM), jnp.float32)
    for b in range(_NUM_BASES):
        sb = s_sum[:, b * _DIM:(b + 1) * _DIM]
        acc = acc + jnp.dot(sb, basis_ref[b],
                            preferred_element_type=jnp.float32)
    out_ref[...] = acc / jnp.maximum(cnt, 1.0)


@jax.jit
def kernel(triplets, unseen_entity, entity_embedding, basis, att):
    trip_t = triplets.astype(jnp.int32).T.reshape(3, _NW, _CHUNK)
    trip_t = jnp.pad(trip_t, ((0, 0), (0, 0), (0, _CHUNK_PAD - _CHUNK)),
                     constant_values=-1)
    u_splat = jnp.full((_LANES,), unseen_entity, dtype=jnp.int32)
    att_t = att.T
    s_all, cnt_all = _sc_kernel(trip_t, u_splat, att_t, entity_embedding)
    out = pl.pallas_call(
        _tc_body,
        out_shape=jax.ShapeDtypeStruct((1, _DIM), jnp.float32),
    )(s_all, cnt_all, basis)
    return out.reshape(_DIM)
